# Initial kernel scaffold; baseline (speedup 1.0000x reference)
import jax
import jax.numpy as jnp
from jax.experimental import pallas as pl


def kernel(x, g, W_self1, W_neigh1, b1, W_self2, W_neigh2, b2):
    raise NotImplementedError("write your pallas kernel here")


# speedup vs baseline: 3.3032x; 3.3032x over previous
import dataclasses

import jax
import jax.numpy as jnp
from jax import lax
from jax.experimental import pallas as pl
from jax.experimental.pallas import tpu as pltpu
from jax.experimental.pallas import tpu_sc as plsc

N_NODES = 10000
N_EDGES = 320000
D = 128

NC = 2
NS = 16
NW = NC * NS
L = 16

E_PAD = 327680
EPT = E_PAD // NW
CH = 128
NCHUNK = EPT // CH
NPAD = 10112
RPT = NPAD // NS

_MESH = plsc.VectorSubcoreMesh(core_axis_name="c", subcore_axis_name="s")


def _sc_segment_sum():
    out_type = [
        jax.ShapeDtypeStruct((NC, NPAD, D), jnp.float32),
        jax.ShapeDtypeStruct((NW * NPAD,), jnp.float32),
    ]
    scratch = [
        pltpu.VMEM((NCHUNK, CH), jnp.int32),
        pltpu.VMEM((NCHUNK, CH), jnp.int32),
        pltpu.VMEM((CH, D), jnp.float32),
        pltpu.VMEM((NPAD,), jnp.float32),
        pltpu.VMEM_SHARED((NPAD, D), jnp.float32),
        pltpu.SemaphoreType.DMA,
    ]

    def body(xw_hbm, src_hbm, dst_hbm, zf_hbm, acc_out, deg_out,
             srcb, dstb, rows, degb, acc_sh, sem):
        cid = lax.axis_index("c")
        sid = lax.axis_index("s")
        wid = cid * NS + sid
        r0 = sid * RPT

        pltpu.sync_copy(zf_hbm.at[pl.ds(r0, RPT)], acc_sh.at[pl.ds(r0, RPT)])
        pltpu.sync_copy(src_hbm.at[wid], srcb)
        pltpu.sync_copy(dst_hbm.at[wid], dstb)

        @pl.loop(0, NPAD // L)
        def _(k):
            degb[pl.ds(k * L, L)] = jnp.zeros((L,), jnp.float32)

        plsc.subcore_barrier()

        ones = jnp.ones((L,), jnp.float32)

        @pl.loop(0, NCHUNK)
        def _(i):
            cp = pltpu.async_copy(xw_hbm.at[srcb.at[i]], rows, sem)

            @pl.loop(0, CH // L)
            def _(j):
                dvec = dstb[i, pl.ds(j * L, L)]
                plsc.addupdate_scatter(degb, [dvec], ones)

            cp.wait()
            pltpu.sync_copy(rows, acc_sh.at[dstb.at[i]], add=True)

        plsc.subcore_barrier()
        pltpu.sync_copy(acc_sh.at[pl.ds(r0, RPT)],
                        acc_out.at[cid, pl.ds(r0, RPT)])
        pltpu.sync_copy(degb, deg_out.at[pl.ds(wid * NPAD, NPAD)])

    cp = pltpu.CompilerParams()
    if "needs_layout_passes" in pltpu.CompilerParams.__dataclass_fields__:
        cp = dataclasses.replace(cp, needs_layout_passes=False)
    return pl.kernel(body, out_type=out_type, mesh=_MESH,
                     scratch_types=scratch, compiler_params=cp)


BR = 1000
DB = 128


def _mm_body(x_ref, w_ref, o1_ref, o2_ref):
    acc = lax.dot_general(
        x_ref[...], w_ref[...], (((1,), (0,)), ((), ())),
        precision=lax.Precision.HIGHEST, preferred_element_type=jnp.float32)
    o1_ref[...] = acc[:, :D]
    o2_ref[...] = acc[:, D:]


def _mm_call(x, wc):
    return pl.pallas_call(
        _mm_body,
        grid=(N_NODES // BR,),
        in_specs=[
            pl.BlockSpec((BR, D), lambda i: (i, 0)),
            pl.BlockSpec((D, 2 * D), lambda i: (0, 0)),
        ],
        out_specs=[
            pl.BlockSpec((BR, D), lambda i: (i, 0)),
            pl.BlockSpec((BR, D), lambda i: (i, 0)),
        ],
        out_shape=[
            jax.ShapeDtypeStruct((N_NODES, D), jnp.float32),
            jax.ShapeDtypeStruct((N_NODES, D), jnp.float32),
        ],
    )(x, wc)


def _deg_reduce_body(dp_ref, o_ref):
    o_ref[...] = jnp.sum(dp_ref[...], axis=0)[:, None]


def _deg_reduce_call(deg_parts):
    return pl.pallas_call(
        _deg_reduce_body,
        grid=(NPAD // DB,),
        in_specs=[pl.BlockSpec((NW, DB), lambda i: (0, i))],
        out_specs=pl.BlockSpec((DB, 1), lambda i: (i, 0)),
        out_shape=jax.ShapeDtypeStruct((NPAD, 1), jnp.float32),
    )(deg_parts)


def _mean_relu(xs, p0, p1, dg, b):
    rdeg = 1.0 / jnp.maximum(dg, 1.0)
    mean = (p0[0] + p1[0]) * rdeg
    return jnp.maximum(xs + mean + b, 0.0)


def _combine_mm_body(xs_ref, p0_ref, p1_ref, dg_ref, b_ref, w_ref,
                     o1_ref, o2_ref):
    h = _mean_relu(xs_ref[...], p0_ref[...], p1_ref[...], dg_ref[...],
                   b_ref[...])
    acc = lax.dot_general(
        h, w_ref[...], (((1,), (0,)), ((), ())),
        precision=lax.Precision.HIGHEST, preferred_element_type=jnp.float32)
    o1_ref[...] = acc[:, :D]
    o2_ref[...] = acc[:, D:]


def _combine_mm_call(xs, pacc, dg, b, wc):
    return pl.pallas_call(
        _combine_mm_body,
        grid=(N_NODES // BR,),
        in_specs=[
            pl.BlockSpec((BR, D), lambda i: (i, 0)),
            pl.BlockSpec((1, BR, D), lambda i: (0, i, 0)),
            pl.BlockSpec((1, BR, D), lambda i: (1, i, 0)),
            pl.BlockSpec((BR, 1), lambda i: (i, 0)),
            pl.BlockSpec((1, D), lambda i: (0, 0)),
            pl.BlockSpec((D, 2 * D), lambda i: (0, 0)),
        ],
        out_specs=[
            pl.BlockSpec((BR, D), lambda i: (i, 0)),
            pl.BlockSpec((BR, D), lambda i: (i, 0)),
        ],
        out_shape=[
            jax.ShapeDtypeStruct((N_NODES, D), jnp.float32),
            jax.ShapeDtypeStruct((N_NODES, D), jnp.float32),
        ],
    )(xs, pacc, pacc, dg, b, wc)


def _final_body(hs_ref, q0_ref, q1_ref, dg_ref, b_ref, o_ref):
    o_ref[...] = _mean_relu(hs_ref[...], q0_ref[...], q1_ref[...],
                            dg_ref[...], b_ref[...])


def _final_call(hs, qacc, dg, b):
    return pl.pallas_call(
        _final_body,
        grid=(N_NODES // BR,),
        in_specs=[
            pl.BlockSpec((BR, D), lambda i: (i, 0)),
            pl.BlockSpec((1, BR, D), lambda i: (0, i, 0)),
            pl.BlockSpec((1, BR, D), lambda i: (1, i, 0)),
            pl.BlockSpec((BR, 1), lambda i: (i, 0)),
            pl.BlockSpec((1, D), lambda i: (0, 0)),
        ],
        out_specs=pl.BlockSpec((BR, D), lambda i: (i, 0)),
        out_shape=jax.ShapeDtypeStruct((N_NODES, D), jnp.float32),
    )(hs, qacc, qacc, dg, b)


def kernel(x, g, W_self1, W_neigh1, b1, W_self2, W_neigh2, b2):
    g32 = g.astype(jnp.int32)
    src3 = jnp.pad(g32[0], (0, E_PAD - N_EDGES)).reshape(NW, NCHUNK, CH)
    dst3 = jnp.pad(g32[1], (0, E_PAD - N_EDGES),
                   constant_values=N_NODES).reshape(NW, NCHUNK, CH)
    zf = jnp.zeros((NPAD, D), jnp.float32)
    w1c = jnp.concatenate([W_neigh1, W_self1], axis=1)
    w2c = jnp.concatenate([W_neigh2, W_self2], axis=1)
    b1r = b1.reshape(1, D)
    b2r = b2.reshape(1, D)

    sc = _sc_segment_sum()
    xwn1, xs1 = _mm_call(x, w1c)
    pacc1, degp = sc(xwn1, src3, dst3, zf)
    dg = _deg_reduce_call(degp.reshape(NW, NPAD))
    hwn2, hs2 = _combine_mm_call(xs1, pacc1, dg, b1r, w2c)
    pacc2, _ = sc(hwn2, src3, dst3, zf)
    return _final_call(hs2, pacc2, dg, b2r)


# speedup vs baseline: 3.3559x; 1.0159x over previous
import dataclasses

import jax
import jax.numpy as jnp
from jax import lax
from jax.experimental import pallas as pl
from jax.experimental.pallas import tpu as pltpu
from jax.experimental.pallas import tpu_sc as plsc

N_NODES = 10000
N_EDGES = 320000
D = 128

NC = 2
NS = 16
NW = NC * NS
L = 16

E_PAD = 327680
EPT = E_PAD // NW
CH = 128
NCHUNK = EPT // CH
NPH = 5
PCH = NCHUNK // NPH
NPAD = 10112
RPT = NPAD // NS

_MESH = plsc.VectorSubcoreMesh(core_axis_name="c", subcore_axis_name="s")


def _sc_segment_sum():
    out_type = [
        jax.ShapeDtypeStruct((NC, NPAD, D), jnp.float32),
        jax.ShapeDtypeStruct((NW * NPAD,), jnp.float32),
    ]
    scratch = [
        pltpu.VMEM((PCH, CH), jnp.int32),
        pltpu.VMEM((PCH, CH), jnp.int32),
        pltpu.VMEM((CH, D), jnp.float32),
        pltpu.VMEM((CH, D), jnp.float32),
        pltpu.VMEM((NPAD,), jnp.float32),
        pltpu.VMEM_SHARED((NPAD, D), jnp.float32),
        pltpu.SemaphoreType.DMA,
        pltpu.SemaphoreType.DMA,
    ]

    def body(xw_hbm, src_hbm, dst_hbm, zf_hbm, acc_out, deg_out,
             srcb, dstb, rows0, rows1, degb, acc_sh, sem0, sem1):
        rows = (rows0, rows1)
        sems = (sem0, sem1)
        cid = lax.axis_index("c")
        sid = lax.axis_index("s")
        wid = cid * NS + sid
        r0 = sid * RPT

        pltpu.sync_copy(zf_hbm.at[pl.ds(r0, RPT)], acc_sh.at[pl.ds(r0, RPT)])

        @pl.loop(0, NPAD // L)
        def _(k):
            degb[pl.ds(k * L, L)] = jnp.zeros((L,), jnp.float32)

        plsc.subcore_barrier()

        ones = jnp.ones((L,), jnp.float32)

        @pl.loop(0, NPH)
        def _(ph):
            c0 = ph * PCH
            pltpu.sync_copy(src_hbm.at[wid, pl.ds(c0, PCH)], srcb)
            pltpu.sync_copy(dst_hbm.at[wid, pl.ds(c0, PCH)], dstb)

            cps = [None] * PCH
            cps[0] = pltpu.async_copy(xw_hbm.at[srcb.at[0]], rows[0], sems[0])
            cps[1] = pltpu.async_copy(xw_hbm.at[srcb.at[1]], rows[1], sems[1])
            for k in range(PCH):
                s = k % 2
                cps[k].wait()
                pltpu.sync_copy(rows[s], acc_sh.at[dstb.at[k]], add=True)
                if k + 2 < PCH:
                    cps[k + 2] = pltpu.async_copy(
                        xw_hbm.at[srcb.at[k + 2]], rows[s], sems[s])

                @pl.loop(0, CH // L)
                def _(j, k=k):
                    dvec = dstb[k, pl.ds(j * L, L)]
                    plsc.addupdate_scatter(degb, [dvec], ones)

        plsc.subcore_barrier()
        pltpu.sync_copy(acc_sh.at[pl.ds(r0, RPT)],
                        acc_out.at[cid, pl.ds(r0, RPT)])
        pltpu.sync_copy(degb, deg_out.at[pl.ds(wid * NPAD, NPAD)])

    cp = pltpu.CompilerParams()
    if "needs_layout_passes" in pltpu.CompilerParams.__dataclass_fields__:
        cp = dataclasses.replace(cp, needs_layout_passes=False)
    return pl.kernel(body, out_type=out_type, mesh=_MESH,
                     scratch_types=scratch, compiler_params=cp)


BR = 1000
DB = 128


def _mm_body(x_ref, w_ref, o1_ref, o2_ref):
    acc = lax.dot_general(
        x_ref[...], w_ref[...], (((1,), (0,)), ((), ())),
        precision=lax.Precision.HIGHEST, preferred_element_type=jnp.float32)
    o1_ref[...] = acc[:, :D]
    o2_ref[...] = acc[:, D:]


def _mm_call(x, wc):
    return pl.pallas_call(
        _mm_body,
        grid=(N_NODES // BR,),
        in_specs=[
            pl.BlockSpec((BR, D), lambda i: (i, 0)),
            pl.BlockSpec((D, 2 * D), lambda i: (0, 0)),
        ],
        out_specs=[
            pl.BlockSpec((BR, D), lambda i: (i, 0)),
            pl.BlockSpec((BR, D), lambda i: (i, 0)),
        ],
        out_shape=[
            jax.ShapeDtypeStruct((N_NODES, D), jnp.float32),
            jax.ShapeDtypeStruct((N_NODES, D), jnp.float32),
        ],
    )(x, wc)


def _deg_reduce_body(dp_ref, o_ref):
    o_ref[...] = jnp.sum(dp_ref[...], axis=0)[:, None]


def _deg_reduce_call(deg_parts):
    return pl.pallas_call(
        _deg_reduce_body,
        grid=(NPAD // DB,),
        in_specs=[pl.BlockSpec((NW, DB), lambda i: (0, i))],
        out_specs=pl.BlockSpec((DB, 1), lambda i: (i, 0)),
        out_shape=jax.ShapeDtypeStruct((NPAD, 1), jnp.float32),
    )(deg_parts)


def _mean_relu(xs, p0, p1, dg, b):
    rdeg = 1.0 / jnp.maximum(dg, 1.0)
    mean = (p0[0] + p1[0]) * rdeg
    return jnp.maximum(xs + mean + b, 0.0)


def _combine_mm_body(xs_ref, p0_ref, p1_ref, dg_ref, b_ref, w_ref,
                     o1_ref, o2_ref):
    h = _mean_relu(xs_ref[...], p0_ref[...], p1_ref[...], dg_ref[...],
                   b_ref[...])
    acc = lax.dot_general(
        h, w_ref[...], (((1,), (0,)), ((), ())),
        precision=lax.Precision.HIGHEST, preferred_element_type=jnp.float32)
    o1_ref[...] = acc[:, :D]
    o2_ref[...] = acc[:, D:]


def _combine_mm_call(xs, pacc, dg, b, wc):
    return pl.pallas_call(
        _combine_mm_body,
        grid=(N_NODES // BR,),
        in_specs=[
            pl.BlockSpec((BR, D), lambda i: (i, 0)),
            pl.BlockSpec((1, BR, D), lambda i: (0, i, 0)),
            pl.BlockSpec((1, BR, D), lambda i: (1, i, 0)),
            pl.BlockSpec((BR, 1), lambda i: (i, 0)),
            pl.BlockSpec((1, D), lambda i: (0, 0)),
            pl.BlockSpec((D, 2 * D), lambda i: (0, 0)),
        ],
        out_specs=[
            pl.BlockSpec((BR, D), lambda i: (i, 0)),
            pl.BlockSpec((BR, D), lambda i: (i, 0)),
        ],
        out_shape=[
            jax.ShapeDtypeStruct((N_NODES, D), jnp.float32),
            jax.ShapeDtypeStruct((N_NODES, D), jnp.float32),
        ],
    )(xs, pacc, pacc, dg, b, wc)


def _final_body(hs_ref, q0_ref, q1_ref, dg_ref, b_ref, o_ref):
    o_ref[...] = _mean_relu(hs_ref[...], q0_ref[...], q1_ref[...],
                            dg_ref[...], b_ref[...])


def _final_call(hs, qacc, dg, b):
    return pl.pallas_call(
        _final_body,
        grid=(N_NODES // BR,),
        in_specs=[
            pl.BlockSpec((BR, D), lambda i: (i, 0)),
            pl.BlockSpec((1, BR, D), lambda i: (0, i, 0)),
            pl.BlockSpec((1, BR, D), lambda i: (1, i, 0)),
            pl.BlockSpec((BR, 1), lambda i: (i, 0)),
            pl.BlockSpec((1, D), lambda i: (0, 0)),
        ],
        out_specs=pl.BlockSpec((BR, D), lambda i: (i, 0)),
        out_shape=jax.ShapeDtypeStruct((N_NODES, D), jnp.float32),
    )(hs, qacc, qacc, dg, b)


def kernel(x, g, W_self1, W_neigh1, b1, W_self2, W_neigh2, b2):
    g32 = g.astype(jnp.int32)
    src3 = jnp.pad(g32[0], (0, E_PAD - N_EDGES)).reshape(NW, NCHUNK, CH)
    dst3 = jnp.pad(g32[1], (0, E_PAD - N_EDGES),
                   constant_values=N_NODES).reshape(NW, NCHUNK, CH)
    zf = jnp.zeros((NPAD, D), jnp.float32)
    w1c = jnp.concatenate([W_neigh1, W_self1], axis=1)
    w2c = jnp.concatenate([W_neigh2, W_self2], axis=1)
    b1r = b1.reshape(1, D)
    b2r = b2.reshape(1, D)

    sc = _sc_segment_sum()
    xwn1, xs1 = _mm_call(x, w1c)
    pacc1, degp = sc(xwn1, src3, dst3, zf)
    dg = _deg_reduce_call(degp.reshape(NW, NPAD))
    hwn2, hs2 = _combine_mm_call(xs1, pacc1, dg, b1r, w2c)
    pacc2, _ = sc(hwn2, src3, dst3, zf)
    return _final_call(hs2, pacc2, dg, b2r)


# speedup vs baseline: 6.9356x; 2.0667x over previous
import dataclasses

import jax
import jax.numpy as jnp
from jax import lax
from jax.experimental import pallas as pl
from jax.experimental.pallas import tpu as pltpu
from jax.experimental.pallas import tpu_sc as plsc

N_NODES = 10000
N_EDGES = 320000
D = 128

NC = 2
NS = 16
NW = NC * NS
L = 16

E_PAD = 327680
EPT = E_PAD // NW
CH = 64
NCHUNK = EPT // CH
NPH = 5
PCH = NCHUNK // NPH
NSLOT = 4
NPAD = 10112
RPT = NPAD // NS

_MESH = plsc.VectorSubcoreMesh(core_axis_name="c", subcore_axis_name="s")


def _compiler_params():
    cp = pltpu.CompilerParams()
    if "needs_layout_passes" in pltpu.CompilerParams.__dataclass_fields__:
        cp = dataclasses.replace(cp, needs_layout_passes=False)
    return cp


def _sc_gather():
    out_type = jax.ShapeDtypeStruct((E_PAD, D), jnp.float32)
    scratch = (
        [pltpu.VMEM((PCH, CH), jnp.int32)]
        + [pltpu.VMEM((CH, D), jnp.float32)] * NSLOT
        + [pltpu.VMEM_SHARED((NPAD, D), jnp.float32)]
        + [pltpu.SemaphoreType.DMA] * (2 * NSLOT)
    )

    def body(xw_hbm, src_hbm, msgs_out, srcb, *rest):
        rows = rest[:NSLOT]
        xw_sh = rest[NSLOT]
        gsems = rest[NSLOT + 1:2 * NSLOT + 1]
        wsems = rest[2 * NSLOT + 1:]
        cid = lax.axis_index("c")
        sid = lax.axis_index("s")
        wid = cid * NS + sid
        r0 = sid * RPT

        pltpu.sync_copy(xw_hbm.at[pl.ds(r0, RPT)], xw_sh.at[pl.ds(r0, RPT)])
        plsc.subcore_barrier()

        e0 = wid * EPT

        @pl.loop(0, NPH)
        def _(ph):
            c0 = ph * PCH
            pltpu.sync_copy(src_hbm.at[wid, pl.ds(c0, PCH)], srcb)

            gcp = [None] * PCH
            wcp = [None] * PCH
            gcp[0] = pltpu.async_copy(xw_sh.at[srcb.at[0]], rows[0],
                                      gsems[0])
            gcp[1] = pltpu.async_copy(xw_sh.at[srcb.at[1]], rows[1],
                                      gsems[1])
            for k in range(PCH):
                j = k % NSLOT
                gcp[k].wait()
                base = e0 + (c0 + k) * CH
                wcp[k] = pltpu.async_copy(
                    rows[j], msgs_out.at[pl.ds(base, CH)], wsems[j])
                if k + 2 < PCH:
                    if k >= 2:
                        wcp[k - 2].wait()
                    j2 = (k + 2) % NSLOT
                    gcp[k + 2] = pltpu.async_copy(
                        xw_sh.at[srcb.at[k + 2]], rows[j2], gsems[j2])

            for t in range(PCH - NSLOT, PCH):
                wcp[t].wait()

    return pl.kernel(body, out_type=out_type, mesh=_MESH,
                     scratch_types=scratch,
                     compiler_params=_compiler_params())


def _sc_scatter():
    out_type = [
        jax.ShapeDtypeStruct((NC, NPAD, D), jnp.float32),
        jax.ShapeDtypeStruct((NW * NPAD,), jnp.float32),
    ]
    scratch = (
        [pltpu.VMEM((PCH, CH), jnp.int32)]
        + [pltpu.VMEM((CH, D), jnp.float32)] * NSLOT
        + [pltpu.VMEM((NPAD,), jnp.float32)]
        + [pltpu.VMEM_SHARED((NPAD, D), jnp.float32)]
        + [pltpu.SemaphoreType.DMA] * (2 * NSLOT)
    )

    def body(msgs_hbm, dst_hbm, zf_hbm, acc_out, deg_out, dstb, *rest):
        rows = rest[:NSLOT]
        degb = rest[NSLOT]
        acc_sh = rest[NSLOT + 1]
        rsems = rest[NSLOT + 2:2 * NSLOT + 2]
        ssems = rest[2 * NSLOT + 2:]
        cid = lax.axis_index("c")
        sid = lax.axis_index("s")
        wid = cid * NS + sid
        r0 = sid * RPT

        pltpu.sync_copy(zf_hbm.at[pl.ds(r0, RPT)], acc_sh.at[pl.ds(r0, RPT)])

        @pl.loop(0, NPAD // L)
        def _(k):
            degb[pl.ds(k * L, L)] = jnp.zeros((L,), jnp.float32)

        plsc.subcore_barrier()

        ones = jnp.ones((L,), jnp.float32)
        e0 = wid * EPT

        @pl.loop(0, NPH)
        def _(ph):
            c0 = ph * PCH
            pltpu.sync_copy(dst_hbm.at[wid, pl.ds(c0, PCH)], dstb)

            rcp = [None] * PCH
            scp = [None] * PCH
            rcp[0] = pltpu.async_copy(
                msgs_hbm.at[pl.ds(e0 + c0 * CH, CH)], rows[0], rsems[0])
            rcp[1] = pltpu.async_copy(
                msgs_hbm.at[pl.ds(e0 + (c0 + 1) * CH, CH)], rows[1],
                rsems[1])
            for k in range(PCH):
                j = k % NSLOT
                rcp[k].wait()
                scp[k] = pltpu.async_copy(rows[j], acc_sh.at[dstb.at[k]],
                                          ssems[j], add=True)
                if k + 2 < PCH:
                    if k >= 2:
                        scp[k - 2].wait()
                    j2 = (k + 2) % NSLOT
                    base = e0 + (c0 + k + 2) * CH
                    rcp[k + 2] = pltpu.async_copy(
                        msgs_hbm.at[pl.ds(base, CH)], rows[j2], rsems[j2])

                @pl.loop(0, CH // L)
                def _(t, k=k):
                    dvec = dstb[k, pl.ds(t * L, L)]
                    plsc.addupdate_scatter(degb, [dvec], ones)

            for t in range(PCH - NSLOT, PCH):
                scp[t].wait()

        plsc.subcore_barrier()
        pltpu.sync_copy(acc_sh.at[pl.ds(r0, RPT)],
                        acc_out.at[cid, pl.ds(r0, RPT)])
        pltpu.sync_copy(degb, deg_out.at[pl.ds(wid * NPAD, NPAD)])

    return pl.kernel(body, out_type=out_type, mesh=_MESH,
                     scratch_types=scratch,
                     compiler_params=_compiler_params())


BR = 1000
DB = 128


def _mm_body(x_ref, w_ref, o1_ref, o2_ref):
    acc = lax.dot_general(
        x_ref[...], w_ref[...], (((1,), (0,)), ((), ())),
        precision=lax.Precision.HIGHEST, preferred_element_type=jnp.float32)
    o1_ref[...] = acc[:, :D]
    o2_ref[...] = acc[:, D:]


def _mm_call(x, wc):
    return pl.pallas_call(
        _mm_body,
        grid=(N_NODES // BR,),
        in_specs=[
            pl.BlockSpec((BR, D), lambda i: (i, 0)),
            pl.BlockSpec((D, 2 * D), lambda i: (0, 0)),
        ],
        out_specs=[
            pl.BlockSpec((BR, D), lambda i: (i, 0)),
            pl.BlockSpec((BR, D), lambda i: (i, 0)),
        ],
        out_shape=[
            jax.ShapeDtypeStruct((N_NODES, D), jnp.float32),
            jax.ShapeDtypeStruct((N_NODES, D), jnp.float32),
        ],
    )(x, wc)


def _deg_reduce_body(dp_ref, o_ref):
    o_ref[...] = jnp.sum(dp_ref[...], axis=0)[:, None]


def _deg_reduce_call(deg_parts):
    return pl.pallas_call(
        _deg_reduce_body,
        grid=(NPAD // DB,),
        in_specs=[pl.BlockSpec((NW, DB), lambda i: (0, i))],
        out_specs=pl.BlockSpec((DB, 1), lambda i: (i, 0)),
        out_shape=jax.ShapeDtypeStruct((NPAD, 1), jnp.float32),
    )(deg_parts)


def _mean_relu(xs, p0, p1, dg, b):
    rdeg = 1.0 / jnp.maximum(dg, 1.0)
    mean = (p0[0] + p1[0]) * rdeg
    return jnp.maximum(xs + mean + b, 0.0)


def _combine_mm_body(xs_ref, p0_ref, p1_ref, dg_ref, b_ref, w_ref,
                     o1_ref, o2_ref):
    h = _mean_relu(xs_ref[...], p0_ref[...], p1_ref[...], dg_ref[...],
                   b_ref[...])
    acc = lax.dot_general(
        h, w_ref[...], (((1,), (0,)), ((), ())),
        precision=lax.Precision.HIGHEST, preferred_element_type=jnp.float32)
    o1_ref[...] = acc[:, :D]
    o2_ref[...] = acc[:, D:]


def _combine_mm_call(xs, pacc, dg, b, wc):
    return pl.pallas_call(
        _combine_mm_body,
        grid=(N_NODES // BR,),
        in_specs=[
            pl.BlockSpec((BR, D), lambda i: (i, 0)),
            pl.BlockSpec((1, BR, D), lambda i: (0, i, 0)),
            pl.BlockSpec((1, BR, D), lambda i: (1, i, 0)),
            pl.BlockSpec((BR, 1), lambda i: (i, 0)),
            pl.BlockSpec((1, D), lambda i: (0, 0)),
            pl.BlockSpec((D, 2 * D), lambda i: (0, 0)),
        ],
        out_specs=[
            pl.BlockSpec((BR, D), lambda i: (i, 0)),
            pl.BlockSpec((BR, D), lambda i: (i, 0)),
        ],
        out_shape=[
            jax.ShapeDtypeStruct((N_NODES, D), jnp.float32),
            jax.ShapeDtypeStruct((N_NODES, D), jnp.float32),
        ],
    )(xs, pacc, pacc, dg, b, wc)


def _final_body(hs_ref, q0_ref, q1_ref, dg_ref, b_ref, o_ref):
    o_ref[...] = _mean_relu(hs_ref[...], q0_ref[...], q1_ref[...],
                            dg_ref[...], b_ref[...])


def _final_call(hs, qacc, dg, b):
    return pl.pallas_call(
        _final_body,
        grid=(N_NODES // BR,),
        in_specs=[
            pl.BlockSpec((BR, D), lambda i: (i, 0)),
            pl.BlockSpec((1, BR, D), lambda i: (0, i, 0)),
            pl.BlockSpec((1, BR, D), lambda i: (1, i, 0)),
            pl.BlockSpec((BR, 1), lambda i: (i, 0)),
            pl.BlockSpec((1, D), lambda i: (0, 0)),
        ],
        out_specs=pl.BlockSpec((BR, D), lambda i: (i, 0)),
        out_shape=jax.ShapeDtypeStruct((N_NODES, D), jnp.float32),
    )(hs, qacc, qacc, dg, b)


def kernel(x, g, W_self1, W_neigh1, b1, W_self2, W_neigh2, b2):
    g32 = g.astype(jnp.int32)
    src3 = jnp.pad(g32[0], (0, E_PAD - N_EDGES)).reshape(NW, NCHUNK, CH)
    dst3 = jnp.pad(g32[1], (0, E_PAD - N_EDGES),
                   constant_values=N_NODES).reshape(NW, NCHUNK, CH)
    zf = jnp.zeros((NPAD, D), jnp.float32)
    w1c = jnp.concatenate([W_neigh1, W_self1], axis=1)
    w2c = jnp.concatenate([W_neigh2, W_self2], axis=1)
    b1r = b1.reshape(1, D)
    b2r = b2.reshape(1, D)
    rpad = ((0, NPAD - N_NODES), (0, 0))

    k1 = _sc_gather()
    k2 = _sc_scatter()
    xwn1, xs1 = _mm_call(x, w1c)
    msgs1 = k1(jnp.pad(xwn1, rpad), src3)
    pacc1, degp = k2(msgs1, dst3, zf)
    dg = _deg_reduce_call(degp.reshape(NW, NPAD))
    hwn2, hs2 = _combine_mm_call(xs1, pacc1, dg, b1r, w2c)
    msgs2 = k1(jnp.pad(hwn2, rpad), src3)
    pacc2, _ = k2(msgs2, dst3, zf)
    return _final_call(hs2, pacc2, dg, b2r)


# speedup vs baseline: 7.2761x; 1.0491x over previous
import dataclasses

import jax
import jax.numpy as jnp
from jax import lax
from jax.experimental import pallas as pl
from jax.experimental.pallas import tpu as pltpu
from jax.experimental.pallas import tpu_sc as plsc

N_NODES = 10000
N_EDGES = 320000
D = 128

NC = 2
NS = 16
NW = NC * NS
L = 16

E_PAD = 327680
EPT = E_PAD // NW
CH = 128
NCHUNK = EPT // CH
NPH = 5
PCH = NCHUNK // NPH
NSLOT = 2
NPAD = 10112
RPT = NPAD // NS

_MESH = plsc.VectorSubcoreMesh(core_axis_name="c", subcore_axis_name="s")


def _compiler_params():
    cp = pltpu.CompilerParams()
    if "needs_layout_passes" in pltpu.CompilerParams.__dataclass_fields__:
        cp = dataclasses.replace(cp, needs_layout_passes=False)
    return cp


def _sc_gather():
    out_type = jax.ShapeDtypeStruct((E_PAD, D), jnp.float32)
    scratch = (
        [pltpu.VMEM((PCH, CH), jnp.int32)]
        + [pltpu.VMEM((CH, D), jnp.float32)] * NSLOT
        + [pltpu.VMEM_SHARED((NPAD, D), jnp.float32)]
        + [pltpu.SemaphoreType.DMA] * (2 * NSLOT)
    )

    def body(xw_hbm, src_hbm, msgs_out, srcb, *rest):
        rows = rest[:NSLOT]
        xw_sh = rest[NSLOT]
        gsems = rest[NSLOT + 1:2 * NSLOT + 1]
        wsems = rest[2 * NSLOT + 1:]
        cid = lax.axis_index("c")
        sid = lax.axis_index("s")
        wid = cid * NS + sid
        r0 = sid * RPT

        pltpu.sync_copy(xw_hbm.at[pl.ds(r0, RPT)], xw_sh.at[pl.ds(r0, RPT)])
        plsc.subcore_barrier()

        e0 = wid * EPT

        @pl.loop(0, NPH)
        def _(ph):
            c0 = ph * PCH
            pltpu.sync_copy(src_hbm.at[wid, pl.ds(c0, PCH)], srcb)

            gcp = [None] * PCH
            wcp = [None] * PCH
            gcp[0] = pltpu.async_copy(xw_sh.at[srcb.at[0]], rows[0],
                                      gsems[0])
            gcp[1] = pltpu.async_copy(xw_sh.at[srcb.at[1]], rows[1],
                                      gsems[1])
            for k in range(PCH):
                j = k % NSLOT
                if k >= 1:
                    wcp[k - 1].wait()
                    if k + 1 < PCH:
                        j1 = (k + 1) % NSLOT
                        gcp[k + 1] = pltpu.async_copy(
                            xw_sh.at[srcb.at[k + 1]], rows[j1], gsems[j1])
                gcp[k].wait()
                base = e0 + (c0 + k) * CH
                wcp[k] = pltpu.async_copy(
                    rows[j], msgs_out.at[pl.ds(base, CH)], wsems[j])

            wcp[PCH - 1].wait()

    return pl.kernel(body, out_type=out_type, mesh=_MESH,
                     scratch_types=scratch,
                     compiler_params=_compiler_params())


def _sc_scatter():
    out_type = [
        jax.ShapeDtypeStruct((NC, NPAD, D), jnp.float32),
        jax.ShapeDtypeStruct((NW * NPAD,), jnp.float32),
    ]
    scratch = (
        [pltpu.VMEM((PCH, CH), jnp.int32)]
        + [pltpu.VMEM((CH, D), jnp.float32)] * NSLOT
        + [pltpu.VMEM((NPAD,), jnp.float32)]
        + [pltpu.VMEM_SHARED((NPAD, D), jnp.float32)]
        + [pltpu.SemaphoreType.DMA] * (2 * NSLOT)
    )

    def body(msgs_hbm, dst_hbm, zf_hbm, acc_out, deg_out, dstb, *rest):
        rows = rest[:NSLOT]
        degb = rest[NSLOT]
        acc_sh = rest[NSLOT + 1]
        rsems = rest[NSLOT + 2:2 * NSLOT + 2]
        ssems = rest[2 * NSLOT + 2:]
        cid = lax.axis_index("c")
        sid = lax.axis_index("s")
        wid = cid * NS + sid
        r0 = sid * RPT

        pltpu.sync_copy(zf_hbm.at[pl.ds(r0, RPT)], acc_sh.at[pl.ds(r0, RPT)])

        @pl.loop(0, NPAD // L)
        def _(k):
            degb[pl.ds(k * L, L)] = jnp.zeros((L,), jnp.float32)

        plsc.subcore_barrier()

        ones = jnp.ones((L,), jnp.float32)
        e0 = wid * EPT

        @pl.loop(0, NPH)
        def _(ph):
            c0 = ph * PCH
            pltpu.sync_copy(dst_hbm.at[wid, pl.ds(c0, PCH)], dstb)

            rcp = [None] * PCH
            scp = [None] * PCH
            rcp[0] = pltpu.async_copy(
                msgs_hbm.at[pl.ds(e0 + c0 * CH, CH)], rows[0], rsems[0])
            rcp[1] = pltpu.async_copy(
                msgs_hbm.at[pl.ds(e0 + (c0 + 1) * CH, CH)], rows[1],
                rsems[1])
            for k in range(PCH):
                j = k % NSLOT
                if k >= 1:
                    scp[k - 1].wait()
                    if k + 1 < PCH:
                        j1 = (k + 1) % NSLOT
                        base = e0 + (c0 + k + 1) * CH
                        rcp[k + 1] = pltpu.async_copy(
                            msgs_hbm.at[pl.ds(base, CH)], rows[j1],
                            rsems[j1])
                rcp[k].wait()
                scp[k] = pltpu.async_copy(rows[j], acc_sh.at[dstb.at[k]],
                                          ssems[j], add=True)

                @pl.loop(0, CH // L)
                def _(t, k=k):
                    dvec = dstb[k, pl.ds(t * L, L)]
                    plsc.addupdate_scatter(degb, [dvec], ones)

            scp[PCH - 1].wait()

        plsc.subcore_barrier()
        pltpu.sync_copy(acc_sh.at[pl.ds(r0, RPT)],
                        acc_out.at[cid, pl.ds(r0, RPT)])
        pltpu.sync_copy(degb, deg_out.at[pl.ds(wid * NPAD, NPAD)])

    return pl.kernel(body, out_type=out_type, mesh=_MESH,
                     scratch_types=scratch,
                     compiler_params=_compiler_params())


BR = 1000
DB = 128


def _mm_body(x_ref, w_ref, o1_ref, o2_ref):
    acc = lax.dot_general(
        x_ref[...], w_ref[...], (((1,), (0,)), ((), ())),
        precision=lax.Precision.HIGHEST, preferred_element_type=jnp.float32)
    o1_ref[...] = acc[:, :D]
    o2_ref[...] = acc[:, D:]


def _mm_call(x, wc):
    return pl.pallas_call(
        _mm_body,
        grid=(N_NODES // BR,),
        in_specs=[
            pl.BlockSpec((BR, D), lambda i: (i, 0)),
            pl.BlockSpec((D, 2 * D), lambda i: (0, 0)),
        ],
        out_specs=[
            pl.BlockSpec((BR, D), lambda i: (i, 0)),
            pl.BlockSpec((BR, D), lambda i: (i, 0)),
        ],
        out_shape=[
            jax.ShapeDtypeStruct((N_NODES, D), jnp.float32),
            jax.ShapeDtypeStruct((N_NODES, D), jnp.float32),
        ],
    )(x, wc)


def _deg_reduce_body(dp_ref, o_ref):
    o_ref[...] = jnp.sum(dp_ref[...], axis=0)[:, None]


def _deg_reduce_call(deg_parts):
    return pl.pallas_call(
        _deg_reduce_body,
        grid=(NPAD // DB,),
        in_specs=[pl.BlockSpec((NW, DB), lambda i: (0, i))],
        out_specs=pl.BlockSpec((DB, 1), lambda i: (i, 0)),
        out_shape=jax.ShapeDtypeStruct((NPAD, 1), jnp.float32),
    )(deg_parts)


def _mean_relu(xs, p0, p1, dg, b):
    rdeg = 1.0 / jnp.maximum(dg, 1.0)
    mean = (p0[0] + p1[0]) * rdeg
    return jnp.maximum(xs + mean + b, 0.0)


def _combine_mm_body(xs_ref, p0_ref, p1_ref, dg_ref, b_ref, w_ref,
                     o1_ref, o2_ref):
    h = _mean_relu(xs_ref[...], p0_ref[...], p1_ref[...], dg_ref[...],
                   b_ref[...])
    acc = lax.dot_general(
        h, w_ref[...], (((1,), (0,)), ((), ())),
        precision=lax.Precision.HIGHEST, preferred_element_type=jnp.float32)
    o1_ref[...] = acc[:, :D]
    o2_ref[...] = acc[:, D:]


def _combine_mm_call(xs, pacc, dg, b, wc):
    return pl.pallas_call(
        _combine_mm_body,
        grid=(N_NODES // BR,),
        in_specs=[
            pl.BlockSpec((BR, D), lambda i: (i, 0)),
            pl.BlockSpec((1, BR, D), lambda i: (0, i, 0)),
            pl.BlockSpec((1, BR, D), lambda i: (1, i, 0)),
            pl.BlockSpec((BR, 1), lambda i: (i, 0)),
            pl.BlockSpec((1, D), lambda i: (0, 0)),
            pl.BlockSpec((D, 2 * D), lambda i: (0, 0)),
        ],
        out_specs=[
            pl.BlockSpec((BR, D), lambda i: (i, 0)),
            pl.BlockSpec((BR, D), lambda i: (i, 0)),
        ],
        out_shape=[
            jax.ShapeDtypeStruct((N_NODES, D), jnp.float32),
            jax.ShapeDtypeStruct((N_NODES, D), jnp.float32),
        ],
    )(xs, pacc, pacc, dg, b, wc)


def _final_body(hs_ref, q0_ref, q1_ref, dg_ref, b_ref, o_ref):
    o_ref[...] = _mean_relu(hs_ref[...], q0_ref[...], q1_ref[...],
                            dg_ref[...], b_ref[...])


def _final_call(hs, qacc, dg, b):
    return pl.pallas_call(
        _final_body,
        grid=(N_NODES // BR,),
        in_specs=[
            pl.BlockSpec((BR, D), lambda i: (i, 0)),
            pl.BlockSpec((1, BR, D), lambda i: (0, i, 0)),
            pl.BlockSpec((1, BR, D), lambda i: (1, i, 0)),
            pl.BlockSpec((BR, 1), lambda i: (i, 0)),
            pl.BlockSpec((1, D), lambda i: (0, 0)),
        ],
        out_specs=pl.BlockSpec((BR, D), lambda i: (i, 0)),
        out_shape=jax.ShapeDtypeStruct((N_NODES, D), jnp.float32),
    )(hs, qacc, qacc, dg, b)


def kernel(x, g, W_self1, W_neigh1, b1, W_self2, W_neigh2, b2):
    g32 = g.astype(jnp.int32)
    src3 = jnp.pad(g32[0], (0, E_PAD - N_EDGES)).reshape(NW, NCHUNK, CH)
    dst3 = jnp.pad(g32[1], (0, E_PAD - N_EDGES),
                   constant_values=N_NODES).reshape(NW, NCHUNK, CH)
    zf = jnp.zeros((NPAD, D), jnp.float32)
    w1c = jnp.concatenate([W_neigh1, W_self1], axis=1)
    w2c = jnp.concatenate([W_neigh2, W_self2], axis=1)
    b1r = b1.reshape(1, D)
    b2r = b2.reshape(1, D)
    rpad = ((0, NPAD - N_NODES), (0, 0))

    k1 = _sc_gather()
    k2 = _sc_scatter()
    xwn1, xs1 = _mm_call(x, w1c)
    msgs1 = k1(jnp.pad(xwn1, rpad), src3)
    pacc1, degp = k2(msgs1, dst3, zf)
    dg = _deg_reduce_call(degp.reshape(NW, NPAD))
    hwn2, hs2 = _combine_mm_call(xs1, pacc1, dg, b1r, w2c)
    msgs2 = k1(jnp.pad(hwn2, rpad), src3)
    pacc2, _ = k2(msgs2, dst3, zf)
    return _final_call(hs2, pacc2, dg, b2r)


# speedup vs baseline: 7.7735x; 1.0684x over previous
import dataclasses

import jax
import jax.numpy as jnp
from jax import lax
from jax.experimental import pallas as pl
from jax.experimental.pallas import tpu as pltpu
from jax.experimental.pallas import tpu_sc as plsc

N_NODES = 10000
N_EDGES = 320000
D = 128

NC = 2
NS = 16
NW = NC * NS
L = 16

E_PAD = 327680
EPT = E_PAD // NW
CH = 128
NCHUNK = EPT // CH
NPH = 5
PCH = NCHUNK // NPH
NSLOT = 2
NPAD = 10112
RPT = NPAD // NS

_MESH = plsc.VectorSubcoreMesh(core_axis_name="c", subcore_axis_name="s")


def _compiler_params():
    cp = pltpu.CompilerParams()
    if "needs_layout_passes" in pltpu.CompilerParams.__dataclass_fields__:
        cp = dataclasses.replace(cp, needs_layout_passes=False)
    return cp


def _sc_gather():
    out_type = jax.ShapeDtypeStruct((E_PAD, D), jnp.float32)
    scratch = (
        [pltpu.VMEM((PCH, CH), jnp.int32)]
        + [pltpu.VMEM((CH, D), jnp.float32)] * NSLOT
        + [pltpu.VMEM_SHARED((NPAD, D), jnp.float32)]
        + [pltpu.SemaphoreType.DMA] * (2 * NSLOT)
    )

    def body(xw_hbm, src_hbm, msgs_out, srcb, *rest):
        rows = rest[:NSLOT]
        xw_sh = rest[NSLOT]
        gsems = rest[NSLOT + 1:2 * NSLOT + 1]
        wsems = rest[2 * NSLOT + 1:]
        cid = lax.axis_index("c")
        sid = lax.axis_index("s")
        wid = cid * NS + sid
        r0 = sid * RPT

        pltpu.sync_copy(xw_hbm.at[pl.ds(r0, RPT)], xw_sh.at[pl.ds(r0, RPT)])
        plsc.subcore_barrier()

        e0 = wid * EPT

        @pl.loop(0, NPH)
        def _(ph):
            c0 = ph * PCH
            pltpu.sync_copy(src_hbm.at[wid, pl.ds(c0, PCH)], srcb)

            gcp = [None] * PCH
            wcp = [None] * PCH
            gcp[0] = pltpu.async_copy(xw_sh.at[srcb.at[0]], rows[0],
                                      gsems[0])
            gcp[1] = pltpu.async_copy(xw_sh.at[srcb.at[1]], rows[1],
                                      gsems[1])
            for k in range(PCH):
                j = k % NSLOT
                if k >= 1:
                    wcp[k - 1].wait()
                    if k + 1 < PCH:
                        j1 = (k + 1) % NSLOT
                        gcp[k + 1] = pltpu.async_copy(
                            xw_sh.at[srcb.at[k + 1]], rows[j1], gsems[j1])
                gcp[k].wait()
                base = e0 + (c0 + k) * CH
                wcp[k] = pltpu.async_copy(
                    rows[j], msgs_out.at[pl.ds(base, CH)], wsems[j])

            wcp[PCH - 1].wait()

    return pl.kernel(body, out_type=out_type, mesh=_MESH,
                     scratch_types=scratch,
                     compiler_params=_compiler_params())


def _sc_scatter():
    out_type = [
        jax.ShapeDtypeStruct((NC, NPAD, D), jnp.float32),
        jax.ShapeDtypeStruct((NW * NPAD,), jnp.float32),
    ]
    scratch = (
        [pltpu.VMEM((PCH, CH), jnp.int32)]
        + [pltpu.VMEM((CH, D), jnp.float32)] * NSLOT
        + [pltpu.VMEM((NPAD,), jnp.float32)]
        + [pltpu.VMEM_SHARED((NPAD, D), jnp.float32)]
        + [pltpu.SemaphoreType.DMA] * (2 * NSLOT)
    )

    def body(msgs_hbm, dst_hbm, zf_hbm, acc_out, deg_out, dstb, *rest):
        rows = rest[:NSLOT]
        degb = rest[NSLOT]
        acc_sh = rest[NSLOT + 1]
        rsems = rest[NSLOT + 2:2 * NSLOT + 2]
        ssems = rest[2 * NSLOT + 2:]
        cid = lax.axis_index("c")
        sid = lax.axis_index("s")
        wid = cid * NS + sid
        r0 = sid * RPT

        pltpu.sync_copy(zf_hbm.at[pl.ds(r0, RPT)], acc_sh.at[pl.ds(r0, RPT)])

        @pl.loop(0, NPAD // L)
        def _(k):
            degb[pl.ds(k * L, L)] = jnp.zeros((L,), jnp.float32)

        plsc.subcore_barrier()

        ones = jnp.ones((L,), jnp.float32)
        e0 = wid * EPT

        @pl.loop(0, NPH)
        def _(ph):
            c0 = ph * PCH
            pltpu.sync_copy(dst_hbm.at[wid, pl.ds(c0, PCH)], dstb)

            rcp = [None] * PCH
            scp = [None] * PCH
            rcp[0] = pltpu.async_copy(
                msgs_hbm.at[pl.ds(e0 + c0 * CH, CH)], rows[0], rsems[0])
            rcp[1] = pltpu.async_copy(
                msgs_hbm.at[pl.ds(e0 + (c0 + 1) * CH, CH)], rows[1],
                rsems[1])
            for k in range(PCH):
                j = k % NSLOT
                if k >= 1:
                    scp[k - 1].wait()
                    if k + 1 < PCH:
                        j1 = (k + 1) % NSLOT
                        base = e0 + (c0 + k + 1) * CH
                        rcp[k + 1] = pltpu.async_copy(
                            msgs_hbm.at[pl.ds(base, CH)], rows[j1],
                            rsems[j1])
                rcp[k].wait()
                scp[k] = pltpu.async_copy(rows[j], acc_sh.at[dstb.at[k]],
                                          ssems[j], add=True)

                @pl.loop(0, CH // L)
                def _(t, k=k):
                    dvec = dstb[k, pl.ds(t * L, L)]
                    plsc.addupdate_scatter(degb, [dvec], ones)

            scp[PCH - 1].wait()

        plsc.subcore_barrier()
        pltpu.sync_copy(acc_sh.at[pl.ds(r0, RPT)],
                        acc_out.at[cid, pl.ds(r0, RPT)])
        pltpu.sync_copy(degb, deg_out.at[pl.ds(wid * NPAD, NPAD)])

    return pl.kernel(body, out_type=out_type, mesh=_MESH,
                     scratch_types=scratch,
                     compiler_params=_compiler_params())


BR = 1000
NDEG = 10240
DB = 1024


def _mm_body(x_ref, w_ref, o1_ref, o2_ref):
    acc = lax.dot_general(
        x_ref[...], w_ref[...], (((1,), (0,)), ((), ())),
        precision=lax.Precision.HIGHEST, preferred_element_type=jnp.float32)
    o1_ref[...] = acc[:, :D]
    o2_ref[...] = acc[:, D:]


def _mm_call(x, wc):
    return pl.pallas_call(
        _mm_body,
        grid=(N_NODES // BR,),
        in_specs=[
            pl.BlockSpec((BR, D), lambda i: (i, 0)),
            pl.BlockSpec((D, 2 * D), lambda i: (0, 0)),
        ],
        out_specs=[
            pl.BlockSpec((BR, D), lambda i: (i, 0)),
            pl.BlockSpec((BR, D), lambda i: (i, 0)),
        ],
        out_shape=[
            jax.ShapeDtypeStruct((N_NODES, D), jnp.float32),
            jax.ShapeDtypeStruct((N_NODES, D), jnp.float32),
        ],
    )(x, wc)


def _deg_reduce_body(dp_ref, o_ref):
    o_ref[...] = jnp.sum(dp_ref[...], axis=0)[:, None]


def _deg_reduce_call(deg_parts):
    return pl.pallas_call(
        _deg_reduce_body,
        grid=(NDEG // DB,),
        in_specs=[pl.BlockSpec((NW, DB), lambda i: (0, i))],
        out_specs=pl.BlockSpec((DB, 1), lambda i: (i, 0)),
        out_shape=jax.ShapeDtypeStruct((NDEG, 1), jnp.float32),
    )(deg_parts)


def _mean_relu(xs, p0, p1, dg, b):
    rdeg = 1.0 / jnp.maximum(dg, 1.0)
    mean = (p0[0] + p1[0]) * rdeg
    return jnp.maximum(xs + mean + b, 0.0)


def _combine_mm_body(xs_ref, p0_ref, p1_ref, dg_ref, b_ref, w_ref,
                     o1_ref, o2_ref):
    h = _mean_relu(xs_ref[...], p0_ref[...], p1_ref[...], dg_ref[...],
                   b_ref[...])
    acc = lax.dot_general(
        h, w_ref[...], (((1,), (0,)), ((), ())),
        precision=lax.Precision.HIGHEST, preferred_element_type=jnp.float32)
    o1_ref[...] = acc[:, :D]
    o2_ref[...] = acc[:, D:]


def _combine_mm_call(xs, pacc, dg, b, wc):
    return pl.pallas_call(
        _combine_mm_body,
        grid=(N_NODES // BR,),
        in_specs=[
            pl.BlockSpec((BR, D), lambda i: (i, 0)),
            pl.BlockSpec((1, BR, D), lambda i: (0, i, 0)),
            pl.BlockSpec((1, BR, D), lambda i: (1, i, 0)),
            pl.BlockSpec((BR, 1), lambda i: (i, 0)),
            pl.BlockSpec((1, D), lambda i: (0, 0)),
            pl.BlockSpec((D, 2 * D), lambda i: (0, 0)),
        ],
        out_specs=[
            pl.BlockSpec((BR, D), lambda i: (i, 0)),
            pl.BlockSpec((BR, D), lambda i: (i, 0)),
        ],
        out_shape=[
            jax.ShapeDtypeStruct((N_NODES, D), jnp.float32),
            jax.ShapeDtypeStruct((N_NODES, D), jnp.float32),
        ],
    )(xs, pacc, pacc, dg, b, wc)


def _final_body(hs_ref, q0_ref, q1_ref, dg_ref, b_ref, o_ref):
    o_ref[...] = _mean_relu(hs_ref[...], q0_ref[...], q1_ref[...],
                            dg_ref[...], b_ref[...])


def _final_call(hs, qacc, dg, b):
    return pl.pallas_call(
        _final_body,
        grid=(N_NODES // BR,),
        in_specs=[
            pl.BlockSpec((BR, D), lambda i: (i, 0)),
            pl.BlockSpec((1, BR, D), lambda i: (0, i, 0)),
            pl.BlockSpec((1, BR, D), lambda i: (1, i, 0)),
            pl.BlockSpec((BR, 1), lambda i: (i, 0)),
            pl.BlockSpec((1, D), lambda i: (0, 0)),
        ],
        out_specs=pl.BlockSpec((BR, D), lambda i: (i, 0)),
        out_shape=jax.ShapeDtypeStruct((N_NODES, D), jnp.float32),
    )(hs, qacc, qacc, dg, b)


def kernel(x, g, W_self1, W_neigh1, b1, W_self2, W_neigh2, b2):
    g32 = g.astype(jnp.int32)
    src3 = jnp.pad(g32[0], (0, E_PAD - N_EDGES)).reshape(NW, NCHUNK, CH)
    dst3 = jnp.pad(g32[1], (0, E_PAD - N_EDGES),
                   constant_values=N_NODES).reshape(NW, NCHUNK, CH)
    zf = jnp.zeros((NPAD, D), jnp.float32)
    w1c = jnp.concatenate([W_neigh1, W_self1], axis=1)
    w2c = jnp.concatenate([W_neigh2, W_self2], axis=1)
    b1r = b1.reshape(1, D)
    b2r = b2.reshape(1, D)
    rpad = ((0, NPAD - N_NODES), (0, 0))

    k1 = _sc_gather()
    k2 = _sc_scatter()
    xwn1, xs1 = _mm_call(x, w1c)
    msgs1 = k1(jnp.pad(xwn1, rpad), src3)
    pacc1, degp = k2(msgs1, dst3, zf)
    dgp = jnp.pad(degp.reshape(NW, NPAD), ((0, 0), (0, NDEG - NPAD)))
    dg = _deg_reduce_call(dgp)
    hwn2, hs2 = _combine_mm_call(xs1, pacc1, dg, b1r, w2c)
    msgs2 = k1(jnp.pad(hwn2, rpad), src3)
    pacc2, _ = k2(msgs2, dst3, zf)
    return _final_call(hs2, pacc2, dg, b2r)
